# baseline (speedup 1.0000x reference)
import jax
import jax.numpy as jnp
from jax import lax
from jax.experimental import pallas as pl
from jax.experimental.pallas import tpu as pltpu
from jax.experimental.pallas import tpu_sc as plsc

BATCH = 16384
EMBED_DIM = 32
PAD = 128

_info = plsc.get_sparse_core_info()
_NC, _NS = _info.num_cores, _info.num_subcores
_NW = _NC * _NS
_B_PER_W = BATCH // _NW


def _body(vi_hbm, vj_hbm, emb_hbm, ctx_hbm, ui_hbm, uj_hbm,
          idx_i_v, idx_j_v, rows_i, rows_j, sem_i, sem_j):
    wid = lax.axis_index("s") * _NC + lax.axis_index("c")
    base = wid * _B_PER_W
    pltpu.sync_copy(vi_hbm.at[pl.ds(base, _B_PER_W)], idx_i_v)
    pltpu.sync_copy(vj_hbm.at[pl.ds(base, _B_PER_W)], idx_j_v)
    cp_i = pltpu.make_async_copy(emb_hbm.at[idx_i_v], rows_i, sem_i)
    cp_j = pltpu.make_async_copy(ctx_hbm.at[idx_j_v], rows_j, sem_j)
    cp_i.start()
    cp_j.start()
    cp_i.wait()
    pltpu.sync_copy(
        rows_i, ui_hbm.at[pl.ds(base, _B_PER_W), pl.ds(0, EMBED_DIM)])
    cp_j.wait()
    pltpu.sync_copy(
        rows_j, uj_hbm.at[pl.ds(base, _B_PER_W), pl.ds(0, EMBED_DIM)])


def kernel(nodeindex, v_i, v_j, embeddings, context_embeddings):
    del nodeindex
    mesh = plsc.VectorSubcoreMesh(core_axis_name="c", subcore_axis_name="s")
    k = pl.kernel(
        _body,
        out_type=(
            jax.ShapeDtypeStruct((BATCH, PAD), jnp.float32),
            jax.ShapeDtypeStruct((BATCH, PAD), jnp.float32),
        ),
        mesh=mesh,
        compiler_params=pltpu.CompilerParams(use_tc_tiling_on_sc=False),
        scratch_types=[
            pltpu.VMEM((_B_PER_W,), jnp.int32),
            pltpu.VMEM((_B_PER_W,), jnp.int32),
            pltpu.VMEM((_B_PER_W, EMBED_DIM), jnp.float32),
            pltpu.VMEM((_B_PER_W, EMBED_DIM), jnp.float32),
            pltpu.SemaphoreType.DMA,
            pltpu.SemaphoreType.DMA,
        ],
    )
    o_i, o_j = k(v_i, v_j, embeddings, context_embeddings)
    return (o_i[:, :EMBED_DIM], o_j[:, :EMBED_DIM])


# speedup vs baseline: 1.4804x; 1.4804x over previous
import jax
import jax.numpy as jnp
from jax import lax
from jax.experimental import pallas as pl
from jax.experimental.pallas import tpu as pltpu
from jax.experimental.pallas import tpu_sc as plsc

BATCH = 16384
EMBED_DIM = 32

_info = plsc.get_sparse_core_info()
_NC, _NS = _info.num_cores, _info.num_subcores
_NW = _NC * _NS
_B_PER_W = BATCH // _NW
_CHUNK = 128
_N_CHUNKS = _B_PER_W // _CHUNK
_L = 16


def _fire(table_hbm, idx_v, buf, sem, cb):
    def grp(g, carry):
        vec = idx_v[pl.ds(cb + g * _L, _L)]
        for l in range(_L):
            pltpu.make_async_copy(
                table_hbm.at[vec[l]], buf.at[g * _L + l], sem).start()
        return carry
    lax.fori_loop(0, _CHUNK // _L, grp, 0)


def _drain(table_hbm, buf, sem):
    pltpu.make_async_copy(table_hbm.at[pl.ds(0, _CHUNK)], buf, sem).wait()


def _body(vi_hbm, vj_hbm, emb_hbm, ctx_hbm, ui_hbm, uj_hbm,
          idx_i_v, idx_j_v, bufs, sems):
    wid = lax.axis_index("s") * _NC + lax.axis_index("c")
    base = wid * _B_PER_W
    pltpu.sync_copy(vi_hbm.at[pl.ds(base, _B_PER_W)], idx_i_v)
    pltpu.sync_copy(vj_hbm.at[pl.ds(base, _B_PER_W)], idx_j_v)
    tables = (emb_hbm, ctx_hbm)
    idxs = (idx_i_v, idx_j_v)
    outs = (ui_hbm, uj_hbm)
    for t in range(2):
        for c in range(2):
            _fire(tables[t], idxs[t], bufs[2 * c + t], sems[2 * c + t],
                  c * _CHUNK)
    for c in range(_N_CHUNKS):
        for t in range(2):
            slot = 2 * (c % 2) + t
            _drain(tables[t], bufs[slot], sems[slot])
            pltpu.sync_copy(bufs[slot],
                            outs[t].at[pl.ds(base + c * _CHUNK, _CHUNK)])
            if c + 2 < _N_CHUNKS:
                _fire(tables[t], idxs[t], bufs[slot], sems[slot],
                      (c + 2) * _CHUNK)


def kernel(nodeindex, v_i, v_j, embeddings, context_embeddings):
    del nodeindex
    mesh = plsc.VectorSubcoreMesh(core_axis_name="c", subcore_axis_name="s")
    k = pl.kernel(
        _body,
        out_type=(
            jax.ShapeDtypeStruct((BATCH, EMBED_DIM), jnp.float32),
            jax.ShapeDtypeStruct((BATCH, EMBED_DIM), jnp.float32),
        ),
        mesh=mesh,
        scratch_types=[
            pltpu.VMEM((_B_PER_W,), jnp.int32),
            pltpu.VMEM((_B_PER_W,), jnp.int32),
            [pltpu.VMEM((_CHUNK, EMBED_DIM), jnp.float32) for _ in range(4)],
            [pltpu.SemaphoreType.DMA for _ in range(4)],
        ],
    )
    u_i, u_j = k(v_i, v_j, embeddings, context_embeddings)
    return (u_i, u_j)
